# baseline (speedup 1.0000x reference)
import functools
import math

import jax
import jax.numpy as jnp
from jax.experimental import pallas as pl

DIM = 768
GROUPS = 8
CG = DIM // GROUPS
HEADS = 8
DIM_HEAD = DIM // HEADS
OFFSET_SCALE = 4.0
OFF_K = 6
DS = 4

_HI = jax.lax.Precision.HIGHEST


def _stage1_kernel(x_ref, wq_ref, wk_ref, wv_ref, w1_ref, b1_ref, w2_ref,
                   qs_ref, k_ref, v_ref, gkv_ref, *, Hs, h2):
    N = Hs * Hs
    KV = h2 * h2
    xr = x_ref[0, 0]
    q = jax.lax.dot(wq_ref[0], xr, precision=_HI)
    scale = DIM_HEAD ** -0.5
    qs_ref[0, 0] = q * scale

    pad = (OFF_K - DS) // 2
    qsq = q.reshape(CG, Hs, Hs)
    qpad = jnp.pad(qsq, ((0, 0), (pad, pad), (pad, pad)))
    w1 = w1_ref[...]
    cols = []
    for oy in range(h2):
        for ox in range(h2):
            patch = qpad[:, DS * oy:DS * oy + OFF_K, DS * ox:DS * ox + OFF_K]
            cols.append(jnp.sum(patch * w1, axis=(1, 2), keepdims=False)[:, None])
    off1 = jnp.concatenate(cols, axis=1) + b1_ref[0][:, None]
    off1 = off1 * 0.5 * (1.0 + jax.lax.erf(off1 * (2.0 ** -0.5)))
    off = jnp.tanh(jax.lax.dot(w2_ref[...], off1, precision=_HI)) * OFFSET_SCALE

    ji = jax.lax.broadcasted_iota(jnp.int32, (1, KV), 1)
    gi = (ji // h2).astype(jnp.float32)
    gj = (ji % h2).astype(jnp.float32)
    vy = gi + off[0:1, :]
    vx = gj + off[1:2, :]
    gy = 2.0 * vy / max(h2 - 1, 1) - 1.0
    gx = 2.0 * vx / max(h2 - 1, 1) - 1.0
    gkv_ref[0, 0] = jnp.concatenate([gx, gy], axis=0)

    ix = ((gx + 1.0) * Hs - 1.0) * 0.5
    iy = ((gy + 1.0) * Hs - 1.0) * 0.5
    ix0 = jnp.floor(ix)
    iy0 = jnp.floor(iy)
    wx1 = ix - ix0
    wx0 = 1.0 - wx1
    wy1 = iy - iy0
    wy0 = 1.0 - wy1
    piota = jax.lax.broadcasted_iota(jnp.int32, (N, KV), 0)
    S = jnp.zeros((N, KV), jnp.float32)
    for cx, cy, wgt in ((ix0, iy0, wx0 * wy0), (ix0 + 1.0, iy0, wx1 * wy0),
                        (ix0, iy0 + 1.0, wx0 * wy1), (ix0 + 1.0, iy0 + 1.0, wx1 * wy1)):
        valid = (cx >= 0) & (cx <= Hs - 1) & (cy >= 0) & (cy <= Hs - 1)
        idx = (jnp.clip(cy, 0, Hs - 1) * Hs + jnp.clip(cx, 0, Hs - 1)).astype(jnp.int32)
        wv_ = jnp.where(valid, wgt, 0.0)
        S = S + jnp.where(piota == idx, 1.0, 0.0) * wv_
    kv = jax.lax.dot(xr, S, precision=_HI)
    k_ref[0, 0] = jax.lax.dot(wk_ref[0], kv, precision=_HI)
    v_ref[0, 0] = jax.lax.dot(wv_ref[0], kv, precision=_HI)


def _stage2_kernel(qs_ref, k_ref, v_ref, gkv_ref, w1t_ref, b1_ref, w2_ref,
                   b2_ref, w3_ref, b3_ref, out_ref, *, Hs, h2, rows):
    N = Hs * Hs
    KV = h2 * h2
    QT = rows * Hs
    T = Hs // rows

    gx = gkv_ref[0, 0, 0:1, :]
    gy = gkv_ref[0, 0, 1:2, :]

    qx = jax.lax.broadcasted_iota(jnp.int32, (Hs, 1), 0).astype(jnp.float32)
    qxn = 2.0 * qx / max(Hs - 1, 1) - 1.0
    fx = None
    w1x = w1t_ref[0:1, :]
    w1y = w1t_ref[1:2, :]
    posx = qxn - gx
    fx = jnp.sign(posx) * jnp.log(jnp.abs(posx) + 1.0)
    R = QT * KV
    ax = fx[:, :, None] * w1x[None, :, :]

    bias_chunks = []
    for t in range(T):
        qy = (jax.lax.broadcasted_iota(jnp.int32, (rows, 1), 0)
              + t * rows).astype(jnp.float32)
        qyn = 2.0 * qy / max(Hs - 1, 1) - 1.0
        posy = qyn - gy
        fy = jnp.sign(posy) * jnp.log(jnp.abs(posy) + 1.0)
        ay = fy[:, :, None] * w1y[None, :, :] + b1_ref[...][None, :, :]
        h1 = jax.nn.relu(ax[None, :, :, :] + ay[:, None, :, :])
        h1 = h1.reshape(R, w2_ref.shape[0])
        h2v = jax.lax.dot_general(h1, w2_ref[...], (((1,), (1,)), ((), ())),
                                  precision=_HI) + b2_ref[...]
        h2v = jax.nn.relu(h2v)
        bc = jax.lax.dot_general(h2v, w3_ref[...], (((1,), (1,)), ((), ())),
                                 precision=_HI)
        bias_chunks.append(bc.reshape(QT, KV))
    bias = jnp.concatenate(bias_chunks, axis=0) + b3_ref[0, 0]

    qb = qs_ref[0, 0]
    kb = k_ref[0, 0]
    vb = v_ref[0, 0]
    sim = jax.lax.dot_general(qb, kb, (((0,), (0,)), ((), ())),
                              precision=_HI) + bias
    m = jnp.max(sim, axis=1, keepdims=True)
    e = jnp.exp(sim - m)
    attn = e / jnp.sum(e, axis=1, keepdims=True)
    out_ref[0, 0] = jax.lax.dot_general(vb, attn, (((1,), (1,)), ((), ())),
                                        precision=_HI)


def _stage3_kernel(oh_ref, wout_ref, bout_ref, y_ref):
    y_ref[0] = (jax.lax.dot(wout_ref[...], oh_ref[0], precision=_HI)
                + bout_ref[0][:, None])


def kernel(x, Wq, Wk, Wv, Woff1, boff1, Woff2, Wcpb1, bcpb1, Wcpb2, bcpb2,
           Wcpb3, bcpb3, Wout, bout):
    B, C, N = x.shape
    Hs = int(math.sqrt(N))
    pad = (OFF_K - DS) // 2
    h2 = (Hs + 2 * pad - OFF_K) // DS + 1
    KV = h2 * h2
    CPB = Wcpb1.shape[0]
    G = GROUPS

    x4 = x.reshape(B, G, CG, N)
    wq3 = Wq[:, :, 0, 0].reshape(G, CG, CG)
    wk3 = Wk[:, :, 0, 0].reshape(G, CG, CG)
    wv3 = Wv[:, :, 0, 0].reshape(G, CG, CG)
    w1 = Woff1[:, 0]
    b1 = boff1.reshape(1, CG)
    w2 = Woff2[:, :, 0, 0]

    s1 = pl.pallas_call(
        functools.partial(_stage1_kernel, Hs=Hs, h2=h2),
        grid=(B, G),
        in_specs=[
            pl.BlockSpec((1, 1, CG, N), lambda b, g: (b, g, 0, 0)),
            pl.BlockSpec((1, CG, CG), lambda b, g: (g, 0, 0)),
            pl.BlockSpec((1, CG, CG), lambda b, g: (g, 0, 0)),
            pl.BlockSpec((1, CG, CG), lambda b, g: (g, 0, 0)),
            pl.BlockSpec((CG, OFF_K, OFF_K), lambda b, g: (0, 0, 0)),
            pl.BlockSpec((1, CG), lambda b, g: (0, 0)),
            pl.BlockSpec((2, CG), lambda b, g: (0, 0)),
        ],
        out_specs=[
            pl.BlockSpec((1, 1, CG, N), lambda b, g: (b, g, 0, 0)),
            pl.BlockSpec((1, 1, CG, KV), lambda b, g: (b, g, 0, 0)),
            pl.BlockSpec((1, 1, CG, KV), lambda b, g: (b, g, 0, 0)),
            pl.BlockSpec((1, 1, 2, KV), lambda b, g: (b, g, 0, 0)),
        ],
        out_shape=[
            jax.ShapeDtypeStruct((B, G, CG, N), jnp.float32),
            jax.ShapeDtypeStruct((B, G, CG, KV), jnp.float32),
            jax.ShapeDtypeStruct((B, G, CG, KV), jnp.float32),
            jax.ShapeDtypeStruct((B, G, 2, KV), jnp.float32),
        ],
    )(x4, wq3, wk3, wv3, w1, b1, w2)
    qs, k, v, gkv = s1

    ROWS = 8
    T = Hs // ROWS
    QT = ROWS * Hs
    w1t = Wcpb1.T
    b1c = bcpb1.reshape(1, CPB)
    b2c = bcpb2.reshape(1, CPB)
    w3 = Wcpb3
    b3 = bcpb3.reshape(1, 1)

    outh = pl.pallas_call(
        functools.partial(_stage2_kernel, Hs=Hs, h2=h2, rows=ROWS),
        grid=(B, G),
        in_specs=[
            pl.BlockSpec((1, 1, CG, N), lambda b, g: (b, g, 0, 0)),
            pl.BlockSpec((1, 1, CG, KV), lambda b, g: (b, g, 0, 0)),
            pl.BlockSpec((1, 1, CG, KV), lambda b, g: (b, g, 0, 0)),
            pl.BlockSpec((1, 1, 2, KV), lambda b, g: (b, g, 0, 0)),
            pl.BlockSpec((2, CPB), lambda b, g: (0, 0)),
            pl.BlockSpec((1, CPB), lambda b, g: (0, 0)),
            pl.BlockSpec((CPB, CPB), lambda b, g: (0, 0)),
            pl.BlockSpec((1, CPB), lambda b, g: (0, 0)),
            pl.BlockSpec((1, CPB), lambda b, g: (0, 0)),
            pl.BlockSpec((1, 1), lambda b, g: (0, 0)),
        ],
        out_specs=pl.BlockSpec((1, 1, CG, N), lambda b, g: (b, g, 0, 0)),
        out_shape=jax.ShapeDtypeStruct((B, G, CG, N), jnp.float32),
    )(qs, k, v, gkv, w1t, b1c, Wcpb2, b2c, w3, b3)

    oh = outh.reshape(B, DIM, N)
    wout2 = Wout[:, :, 0, 0]
    y = pl.pallas_call(
        _stage3_kernel,
        grid=(B,),
        in_specs=[
            pl.BlockSpec((1, DIM, N), lambda b: (b, 0, 0)),
            pl.BlockSpec((DIM, DIM), lambda b: (0, 0)),
            pl.BlockSpec((1, DIM), lambda b: (0, 0)),
        ],
        out_specs=pl.BlockSpec((1, DIM, N), lambda b: (b, 0, 0)),
        out_shape=jax.ShapeDtypeStruct((B, DIM, N), jnp.float32),
    )(oh, wout2, bout.reshape(1, DIM))
    return y.reshape(B, DIM, Hs, Hs)


# speedup vs baseline: 2.0703x; 2.0703x over previous
import functools
import math

import jax
import jax.numpy as jnp
from jax.experimental import pallas as pl
from jax.experimental.pallas import tpu as pltpu

DIM = 768
GROUPS = 8
CG = DIM // GROUPS
HEADS = 8
DIM_HEAD = DIM // HEADS
OFFSET_SCALE = 4.0
OFF_K = 6
DS = 4

_HI = jax.lax.Precision.HIGHEST


def _stage1_kernel(x_ref, wq_ref, wk_ref, wv_ref, w1_ref, b1_ref, w2_ref,
                   qs_ref, k_ref, v_ref, gkv_ref, *, Hs, h2):
    N = Hs * Hs
    KV = h2 * h2
    xr = x_ref[0, 0]
    q = jax.lax.dot(wq_ref[0], xr, precision=_HI)
    scale = DIM_HEAD ** -0.5
    qs_ref[0, 0] = q * scale

    pad = (OFF_K - DS) // 2
    qsq = q.reshape(CG, Hs, Hs)
    qpad = jnp.pad(qsq, ((0, 0), (pad, pad), (pad, pad)))
    w1 = w1_ref[...]
    cols = []
    for oy in range(h2):
        for ox in range(h2):
            patch = qpad[:, DS * oy:DS * oy + OFF_K, DS * ox:DS * ox + OFF_K]
            cols.append(jnp.sum(patch * w1, axis=(1, 2), keepdims=False)[:, None])
    off1 = jnp.concatenate(cols, axis=1) + b1_ref[0][:, None]
    off1 = off1 * 0.5 * (1.0 + jax.lax.erf(off1 * (2.0 ** -0.5)))
    off = jnp.tanh(jax.lax.dot(w2_ref[...], off1, precision=_HI)) * OFFSET_SCALE

    ji = jax.lax.broadcasted_iota(jnp.int32, (1, KV), 1)
    gi = (ji // h2).astype(jnp.float32)
    gj = (ji % h2).astype(jnp.float32)
    vy = gi + off[0:1, :]
    vx = gj + off[1:2, :]
    gy = 2.0 * vy / max(h2 - 1, 1) - 1.0
    gx = 2.0 * vx / max(h2 - 1, 1) - 1.0
    gkv_ref[0, 0] = jnp.concatenate([gx, gy], axis=0)

    ix = ((gx + 1.0) * Hs - 1.0) * 0.5
    iy = ((gy + 1.0) * Hs - 1.0) * 0.5
    ix0 = jnp.floor(ix)
    iy0 = jnp.floor(iy)
    wx1 = ix - ix0
    wx0 = 1.0 - wx1
    wy1 = iy - iy0
    wy0 = 1.0 - wy1
    piota = jax.lax.broadcasted_iota(jnp.int32, (N, KV), 0)
    S = jnp.zeros((N, KV), jnp.float32)
    for cx, cy, wgt in ((ix0, iy0, wx0 * wy0), (ix0 + 1.0, iy0, wx1 * wy0),
                        (ix0, iy0 + 1.0, wx0 * wy1), (ix0 + 1.0, iy0 + 1.0, wx1 * wy1)):
        valid = (cx >= 0) & (cx <= Hs - 1) & (cy >= 0) & (cy <= Hs - 1)
        idx = (jnp.clip(cy, 0, Hs - 1) * Hs + jnp.clip(cx, 0, Hs - 1)).astype(jnp.int32)
        wv_ = jnp.where(valid, wgt, 0.0)
        S = S + jnp.where(piota == idx, 1.0, 0.0) * wv_
    kv = jax.lax.dot(xr, S, precision=_HI)
    k_ref[0, 0] = jax.lax.dot(wk_ref[0], kv, precision=_HI)
    v_ref[0, 0] = jax.lax.dot(wv_ref[0], kv, precision=_HI)


def _stage2_kernel(qs_ref, k_ref, v_ref, gkv_ref, w1t_ref, b1_ref, w2_ref,
                   b2_ref, w3_ref, b3_ref, out_ref, *, Hs, h2, rows):
    N = Hs * Hs
    KV = h2 * h2
    QT = rows * Hs
    T = Hs // rows

    gx = gkv_ref[0, 0, 0:1, :]
    gy = gkv_ref[0, 0, 1:2, :]

    qx = jax.lax.broadcasted_iota(jnp.int32, (Hs, 1), 0).astype(jnp.float32)
    qxn = 2.0 * qx / max(Hs - 1, 1) - 1.0
    fx = None
    w1x = w1t_ref[0:1, :]
    w1y = w1t_ref[1:2, :]
    posx = qxn - gx
    fx = jnp.sign(posx) * jnp.log(jnp.abs(posx) + 1.0)
    R = QT * KV
    ax = fx[:, :, None] * w1x[None, :, :]

    bias_chunks = []
    for t in range(T):
        qy = (jax.lax.broadcasted_iota(jnp.int32, (rows, 1), 0)
              + t * rows).astype(jnp.float32)
        qyn = 2.0 * qy / max(Hs - 1, 1) - 1.0
        posy = qyn - gy
        fy = jnp.sign(posy) * jnp.log(jnp.abs(posy) + 1.0)
        ay = fy[:, :, None] * w1y[None, :, :] + b1_ref[...][None, :, :]
        h1 = jax.nn.relu(ax[None, :, :, :] + ay[:, None, :, :])
        h1 = h1.reshape(R, w2_ref.shape[0]).astype(jnp.bfloat16)
        h2v = jax.lax.dot_general(h1, w2_ref[...], (((1,), (1,)), ((), ())),
                                  preferred_element_type=jnp.float32) + b2_ref[...]
        h2v = jax.nn.relu(h2v)
        bc = jnp.sum(h2v * w3_ref[...], axis=1, keepdims=True)
        bias_chunks.append(bc.reshape(QT, KV))
    bias = jnp.concatenate(bias_chunks, axis=0) + b3_ref[0, 0]

    qb = qs_ref[0, 0]
    kb = k_ref[0, 0]
    vb = v_ref[0, 0]
    sim = jax.lax.dot_general(qb, kb, (((0,), (0,)), ((), ())),
                              precision=_HI) + bias
    m = jnp.max(sim, axis=1, keepdims=True)
    e = jnp.exp(sim - m)
    attn = e / jnp.sum(e, axis=1, keepdims=True)
    out_ref[0, 0] = jax.lax.dot_general(vb, attn, (((1,), (1,)), ((), ())),
                                        precision=_HI)


def _stage3_kernel(oh_ref, wout_ref, bout_ref, y_ref):
    y_ref[0] = (jax.lax.dot(wout_ref[...], oh_ref[0], precision=_HI)
                + bout_ref[0][:, None])


def kernel(x, Wq, Wk, Wv, Woff1, boff1, Woff2, Wcpb1, bcpb1, Wcpb2, bcpb2,
           Wcpb3, bcpb3, Wout, bout):
    B, C, N = x.shape
    Hs = int(math.sqrt(N))
    pad = (OFF_K - DS) // 2
    h2 = (Hs + 2 * pad - OFF_K) // DS + 1
    KV = h2 * h2
    CPB = Wcpb1.shape[0]
    G = GROUPS

    x4 = x.reshape(B, G, CG, N)
    wq3 = Wq[:, :, 0, 0].reshape(G, CG, CG)
    wk3 = Wk[:, :, 0, 0].reshape(G, CG, CG)
    wv3 = Wv[:, :, 0, 0].reshape(G, CG, CG)
    w1 = Woff1[:, 0]
    b1 = boff1.reshape(1, CG)
    w2 = Woff2[:, :, 0, 0]

    s1 = pl.pallas_call(
        functools.partial(_stage1_kernel, Hs=Hs, h2=h2),
        grid=(B, G),
        in_specs=[
            pl.BlockSpec((1, 1, CG, N), lambda b, g: (b, g, 0, 0)),
            pl.BlockSpec((1, CG, CG), lambda b, g: (g, 0, 0)),
            pl.BlockSpec((1, CG, CG), lambda b, g: (g, 0, 0)),
            pl.BlockSpec((1, CG, CG), lambda b, g: (g, 0, 0)),
            pl.BlockSpec((CG, OFF_K, OFF_K), lambda b, g: (0, 0, 0)),
            pl.BlockSpec((1, CG), lambda b, g: (0, 0)),
            pl.BlockSpec((2, CG), lambda b, g: (0, 0)),
        ],
        out_specs=[
            pl.BlockSpec((1, 1, CG, N), lambda b, g: (b, g, 0, 0)),
            pl.BlockSpec((1, 1, CG, KV), lambda b, g: (b, g, 0, 0)),
            pl.BlockSpec((1, 1, CG, KV), lambda b, g: (b, g, 0, 0)),
            pl.BlockSpec((1, 1, 2, KV), lambda b, g: (b, g, 0, 0)),
        ],
        out_shape=[
            jax.ShapeDtypeStruct((B, G, CG, N), jnp.float32),
            jax.ShapeDtypeStruct((B, G, CG, KV), jnp.float32),
            jax.ShapeDtypeStruct((B, G, CG, KV), jnp.float32),
            jax.ShapeDtypeStruct((B, G, 2, KV), jnp.float32),
        ],
        compiler_params=pltpu.CompilerParams(
            dimension_semantics=("parallel", "parallel")),
    )(x4, wq3, wk3, wv3, w1, b1, w2)
    qs, k, v, gkv = s1

    ROWS = 8
    T = Hs // ROWS
    QT = ROWS * Hs
    w1t = Wcpb1.T
    b1c = bcpb1.reshape(1, CPB)
    b2c = bcpb2.reshape(1, CPB)
    w2b = Wcpb2.astype(jnp.bfloat16)
    w3 = Wcpb3
    b3 = bcpb3.reshape(1, 1)

    outh = pl.pallas_call(
        functools.partial(_stage2_kernel, Hs=Hs, h2=h2, rows=ROWS),
        grid=(B, G),
        in_specs=[
            pl.BlockSpec((1, 1, CG, N), lambda b, g: (b, g, 0, 0)),
            pl.BlockSpec((1, 1, CG, KV), lambda b, g: (b, g, 0, 0)),
            pl.BlockSpec((1, 1, CG, KV), lambda b, g: (b, g, 0, 0)),
            pl.BlockSpec((1, 1, 2, KV), lambda b, g: (b, g, 0, 0)),
            pl.BlockSpec((2, CPB), lambda b, g: (0, 0)),
            pl.BlockSpec((1, CPB), lambda b, g: (0, 0)),
            pl.BlockSpec((CPB, CPB), lambda b, g: (0, 0)),
            pl.BlockSpec((1, CPB), lambda b, g: (0, 0)),
            pl.BlockSpec((1, CPB), lambda b, g: (0, 0)),
            pl.BlockSpec((1, 1), lambda b, g: (0, 0)),
        ],
        out_specs=pl.BlockSpec((1, 1, CG, N), lambda b, g: (b, g, 0, 0)),
        out_shape=jax.ShapeDtypeStruct((B, G, CG, N), jnp.float32),
        compiler_params=pltpu.CompilerParams(
            dimension_semantics=("parallel", "parallel")),
    )(qs, k, v, gkv, w1t, b1c, w2b, b2c, w3, b3)

    oh = outh.reshape(B, DIM, N)
    wout2 = Wout[:, :, 0, 0]
    y = pl.pallas_call(
        _stage3_kernel,
        grid=(B,),
        in_specs=[
            pl.BlockSpec((1, DIM, N), lambda b: (b, 0, 0)),
            pl.BlockSpec((DIM, DIM), lambda b: (0, 0)),
            pl.BlockSpec((1, DIM), lambda b: (0, 0)),
        ],
        out_specs=pl.BlockSpec((1, DIM, N), lambda b: (b, 0, 0)),
        out_shape=jax.ShapeDtypeStruct((B, DIM, N), jnp.float32),
        compiler_params=pltpu.CompilerParams(
            dimension_semantics=("parallel",)),
    )(oh, wout2, bout.reshape(1, DIM))
    return y.reshape(B, DIM, Hs, Hs)


# speedup vs baseline: 4.7018x; 2.2711x over previous
import functools
import math

import jax
import jax.numpy as jnp
from jax.experimental import pallas as pl
from jax.experimental.pallas import tpu as pltpu

DIM = 768
GROUPS = 8
CG = DIM // GROUPS
HEADS = 8
DIM_HEAD = DIM // HEADS
OFFSET_SCALE = 4.0
OFF_K = 6
DS = 4

_HI = jax.lax.Precision.HIGHEST


def _stage1_kernel(x_ref, wq_ref, wk_ref, wv_ref, w1_ref, b1_ref, w2_ref,
                   qs_ref, k_ref, v_ref, gkv_ref, *, Hs, h2):
    N = Hs * Hs
    KV = h2 * h2
    xr = x_ref[0, 0]
    q = jax.lax.dot(wq_ref[0], xr, precision=_HI)
    scale = DIM_HEAD ** -0.5
    qs_ref[0, 0] = q * scale

    pad = (OFF_K - DS) // 2
    qT = jax.lax.dot_general(xr, wq_ref[0], (((0,), (1,)), ((), ())),
                             precision=_HI)
    q3 = qT.reshape(Hs, Hs, CG)
    rpad_hi = DS * (h2 - 1) + OFF_K - pad - Hs
    zc = jnp.zeros((pad, Hs, CG), jnp.float32)
    zc2 = jnp.zeros((rpad_hi, Hs, CG), jnp.float32)
    qp = jnp.concatenate([zc, q3, zc2], axis=0)
    Hp = Hs + pad + rpad_hi
    zr = jnp.zeros((Hp, pad, CG), jnp.float32)
    zr2 = jnp.zeros((Hp, rpad_hi, CG), jnp.float32)
    qp = jnp.concatenate([zr, qp, zr2], axis=1)
    w1 = w1_ref[...]
    rows_o = []
    for oy in range(h2):
        for ox in range(h2):
            patch = qp[DS * oy:DS * oy + OFF_K, DS * ox:DS * ox + OFF_K, :]
            rows_o.append(jnp.sum(patch * w1, axis=(0, 1), keepdims=True
                                  ).reshape(1, CG))
    off1 = jnp.concatenate(rows_o, axis=0) + b1_ref[...]
    off1 = off1 * 0.5 * (1.0 + jax.lax.erf(off1 * (2.0 ** -0.5)))
    off = jnp.tanh(jax.lax.dot_general(w2_ref[...], off1, (((1,), (1,)), ((), ())),
                                       precision=_HI)) * OFFSET_SCALE

    ji = jax.lax.broadcasted_iota(jnp.int32, (1, KV), 1)
    gi = (ji // h2).astype(jnp.float32)
    gj = (ji % h2).astype(jnp.float32)
    vy = gi + off[0:1, :]
    vx = gj + off[1:2, :]
    gy = 2.0 * vy / max(h2 - 1, 1) - 1.0
    gx = 2.0 * vx / max(h2 - 1, 1) - 1.0
    gkv_ref[0, 0] = jnp.concatenate([gx, gy], axis=0)

    ix = ((gx + 1.0) * Hs - 1.0) * 0.5
    iy = ((gy + 1.0) * Hs - 1.0) * 0.5
    ix0 = jnp.floor(ix)
    iy0 = jnp.floor(iy)
    wx1 = ix - ix0
    wx0 = 1.0 - wx1
    wy1 = iy - iy0
    wy0 = 1.0 - wy1
    piota = jax.lax.broadcasted_iota(jnp.int32, (N, KV), 0)
    S = jnp.zeros((N, KV), jnp.float32)
    for cx, cy, wgt in ((ix0, iy0, wx0 * wy0), (ix0 + 1.0, iy0, wx1 * wy0),
                        (ix0, iy0 + 1.0, wx0 * wy1), (ix0 + 1.0, iy0 + 1.0, wx1 * wy1)):
        valid = (cx >= 0) & (cx <= Hs - 1) & (cy >= 0) & (cy <= Hs - 1)
        idx = (jnp.clip(cy, 0, Hs - 1) * Hs + jnp.clip(cx, 0, Hs - 1)).astype(jnp.int32)
        wv_ = jnp.where(valid, wgt, 0.0)
        S = S + jnp.where(piota == idx, 1.0, 0.0) * wv_
    kv = jax.lax.dot(xr, S, precision=_HI)
    k_ref[0, 0] = jax.lax.dot(wk_ref[0], kv, precision=_HI)
    v_ref[0, 0] = jax.lax.dot(wv_ref[0], kv, precision=_HI)


def _stage2_kernel(qs_ref, k_ref, v_ref, gkv_ref, w1t_ref, b1_ref, w2_ref,
                   b2_ref, w3_ref, b3_ref, out_ref, *, Hs, h2, rows):
    N = Hs * Hs
    KV = h2 * h2
    QT = rows * Hs
    T = Hs // rows

    gx = gkv_ref[0, 0, 0:1, :]
    gy = gkv_ref[0, 0, 1:2, :]

    qx = jax.lax.broadcasted_iota(jnp.int32, (Hs, 1), 0).astype(jnp.float32)
    qxn = 2.0 * qx / max(Hs - 1, 1) - 1.0
    fx = None
    w1x = w1t_ref[0:1, :]
    w1y = w1t_ref[1:2, :]
    posx = qxn - gx
    fx = jnp.sign(posx) * jnp.log(jnp.abs(posx) + 1.0)
    R = QT * KV
    ax = fx[:, :, None] * w1x[None, :, :]

    bias_chunks = []
    for t in range(T):
        qy = (jax.lax.broadcasted_iota(jnp.int32, (rows, 1), 0)
              + t * rows).astype(jnp.float32)
        qyn = 2.0 * qy / max(Hs - 1, 1) - 1.0
        posy = qyn - gy
        fy = jnp.sign(posy) * jnp.log(jnp.abs(posy) + 1.0)
        ay = fy[:, :, None] * w1y[None, :, :] + b1_ref[...][None, :, :]
        h1 = jax.nn.relu(ax[None, :, :, :] + ay[:, None, :, :])
        h1 = h1.reshape(R, w2_ref.shape[0]).astype(jnp.bfloat16)
        h2v = jax.lax.dot_general(h1, w2_ref[...], (((1,), (1,)), ((), ())),
                                  preferred_element_type=jnp.float32) + b2_ref[...]
        h2v = jax.nn.relu(h2v).astype(jnp.bfloat16)
        bc = jax.lax.dot_general(h2v, w3_ref[...], (((1,), (0,)), ((), ())),
                                 preferred_element_type=jnp.float32)
        bias_chunks.append(bc[:, 0:1].reshape(QT, KV))
    bias = jnp.concatenate(bias_chunks, axis=0) + b3_ref[0, 0]

    qb = qs_ref[0, 0]
    kb = k_ref[0, 0]
    vb = v_ref[0, 0]
    sim = jax.lax.dot_general(qb, kb, (((0,), (0,)), ((), ())),
                              precision=_HI) + bias
    m = jnp.max(sim, axis=1, keepdims=True)
    e = jnp.exp(sim - m)
    attn = e / jnp.sum(e, axis=1, keepdims=True)
    out_ref[0, 0] = jax.lax.dot_general(vb, attn, (((1,), (1,)), ((), ())),
                                        precision=_HI)


def _stage3_kernel(oh_ref, wout_ref, bout_ref, y_ref):
    y_ref[0] = (jax.lax.dot(wout_ref[...], oh_ref[0], precision=_HI)
                + bout_ref[0][:, None])


def kernel(x, Wq, Wk, Wv, Woff1, boff1, Woff2, Wcpb1, bcpb1, Wcpb2, bcpb2,
           Wcpb3, bcpb3, Wout, bout):
    B, C, N = x.shape
    Hs = int(math.sqrt(N))
    pad = (OFF_K - DS) // 2
    h2 = (Hs + 2 * pad - OFF_K) // DS + 1
    KV = h2 * h2
    CPB = Wcpb1.shape[0]
    G = GROUPS

    x4 = x.reshape(B, G, CG, N)
    wq3 = Wq[:, :, 0, 0].reshape(G, CG, CG)
    wk3 = Wk[:, :, 0, 0].reshape(G, CG, CG)
    wv3 = Wv[:, :, 0, 0].reshape(G, CG, CG)
    w1 = Woff1[:, 0].transpose(1, 2, 0)
    b1 = boff1.reshape(1, CG)
    w2 = Woff2[:, :, 0, 0]

    s1 = pl.pallas_call(
        functools.partial(_stage1_kernel, Hs=Hs, h2=h2),
        grid=(B, G),
        in_specs=[
            pl.BlockSpec((1, 1, CG, N), lambda b, g: (b, g, 0, 0)),
            pl.BlockSpec((1, CG, CG), lambda b, g: (g, 0, 0)),
            pl.BlockSpec((1, CG, CG), lambda b, g: (g, 0, 0)),
            pl.BlockSpec((1, CG, CG), lambda b, g: (g, 0, 0)),
            pl.BlockSpec((OFF_K, OFF_K, CG), lambda b, g: (0, 0, 0)),
            pl.BlockSpec((1, CG), lambda b, g: (0, 0)),
            pl.BlockSpec((2, CG), lambda b, g: (0, 0)),
        ],
        out_specs=[
            pl.BlockSpec((1, 1, CG, N), lambda b, g: (b, g, 0, 0)),
            pl.BlockSpec((1, 1, CG, KV), lambda b, g: (b, g, 0, 0)),
            pl.BlockSpec((1, 1, CG, KV), lambda b, g: (b, g, 0, 0)),
            pl.BlockSpec((1, 1, 2, KV), lambda b, g: (b, g, 0, 0)),
        ],
        out_shape=[
            jax.ShapeDtypeStruct((B, G, CG, N), jnp.float32),
            jax.ShapeDtypeStruct((B, G, CG, KV), jnp.float32),
            jax.ShapeDtypeStruct((B, G, CG, KV), jnp.float32),
            jax.ShapeDtypeStruct((B, G, 2, KV), jnp.float32),
        ],
        compiler_params=pltpu.CompilerParams(
            dimension_semantics=("parallel", "parallel")),
    )(x4, wq3, wk3, wv3, w1, b1, w2)
    qs, k, v, gkv = s1

    ROWS = 8
    T = Hs // ROWS
    QT = ROWS * Hs
    w1t = Wcpb1.T
    b1c = bcpb1.reshape(1, CPB)
    b2c = bcpb2.reshape(1, CPB)
    w2b = Wcpb2.astype(jnp.bfloat16)
    w3 = jnp.zeros((CPB, 128), jnp.float32).at[:, 0].set(
        Wcpb3[0]).astype(jnp.bfloat16)
    b3 = bcpb3.reshape(1, 1)

    outh = pl.pallas_call(
        functools.partial(_stage2_kernel, Hs=Hs, h2=h2, rows=ROWS),
        grid=(B, G),
        in_specs=[
            pl.BlockSpec((1, 1, CG, N), lambda b, g: (b, g, 0, 0)),
            pl.BlockSpec((1, 1, CG, KV), lambda b, g: (b, g, 0, 0)),
            pl.BlockSpec((1, 1, CG, KV), lambda b, g: (b, g, 0, 0)),
            pl.BlockSpec((1, 1, 2, KV), lambda b, g: (b, g, 0, 0)),
            pl.BlockSpec((2, CPB), lambda b, g: (0, 0)),
            pl.BlockSpec((1, CPB), lambda b, g: (0, 0)),
            pl.BlockSpec((CPB, CPB), lambda b, g: (0, 0)),
            pl.BlockSpec((1, CPB), lambda b, g: (0, 0)),
            pl.BlockSpec((CPB, 128), lambda b, g: (0, 0)),
            pl.BlockSpec((1, 1), lambda b, g: (0, 0)),
        ],
        out_specs=pl.BlockSpec((1, 1, CG, N), lambda b, g: (b, g, 0, 0)),
        out_shape=jax.ShapeDtypeStruct((B, G, CG, N), jnp.float32),
        compiler_params=pltpu.CompilerParams(
            dimension_semantics=("parallel", "parallel")),
    )(qs, k, v, gkv, w1t, b1c, w2b, b2c, w3, b3)

    oh = outh.reshape(B, DIM, N)
    wout2 = Wout[:, :, 0, 0]
    y = pl.pallas_call(
        _stage3_kernel,
        grid=(B,),
        in_specs=[
            pl.BlockSpec((1, DIM, N), lambda b: (b, 0, 0)),
            pl.BlockSpec((DIM, DIM), lambda b: (0, 0)),
            pl.BlockSpec((1, DIM), lambda b: (0, 0)),
        ],
        out_specs=pl.BlockSpec((1, DIM, N), lambda b: (b, 0, 0)),
        out_shape=jax.ShapeDtypeStruct((B, DIM, N), jnp.float32),
        compiler_params=pltpu.CompilerParams(
            dimension_semantics=("parallel",)),
    )(oh, wout2, bout.reshape(1, DIM))
    return y.reshape(B, DIM, Hs, Hs)


# speedup vs baseline: 4.8669x; 1.0351x over previous
import functools
import math

import jax
import jax.numpy as jnp
from jax.experimental import pallas as pl
from jax.experimental.pallas import tpu as pltpu

DIM = 768
GROUPS = 8
CG = DIM // GROUPS
HEADS = 8
DIM_HEAD = DIM // HEADS
OFFSET_SCALE = 4.0
OFF_K = 6
DS = 4

_HI = jax.lax.Precision.HIGHEST


def _stageA_kernel(x_ref, wq_ref, wk_ref, wv_ref, w1_ref, b1_ref, w2_ref,
                   w1t_ref, b1c_ref, w2c_ref, b2c_ref, w3c_ref, b3c_ref,
                   out_ref, *, Hs, h2, rows):
    N = Hs * Hs
    KV = h2 * h2
    CPB = w2c_ref.shape[0]
    QT = rows * Hs
    T = Hs // rows
    xr = x_ref[0, 0]

    qT = jax.lax.dot_general(xr, wq_ref[0], (((0,), (1,)), ((), ())),
                             precision=_HI)

    pad = (OFF_K - DS) // 2
    q3 = qT.reshape(Hs, Hs, CG)
    rpad_hi = DS * (h2 - 1) + OFF_K - pad - Hs
    zc = jnp.zeros((pad, Hs, CG), jnp.float32)
    zc2 = jnp.zeros((rpad_hi, Hs, CG), jnp.float32)
    qp = jnp.concatenate([zc, q3, zc2], axis=0)
    Hp = Hs + pad + rpad_hi
    zr = jnp.zeros((Hp, pad, CG), jnp.float32)
    zr2 = jnp.zeros((Hp, rpad_hi, CG), jnp.float32)
    qp = jnp.concatenate([zr, qp, zr2], axis=1)
    w1 = w1_ref[...]
    rows_o = []
    for oy in range(h2):
        for ox in range(h2):
            patch = qp[DS * oy:DS * oy + OFF_K, DS * ox:DS * ox + OFF_K, :]
            rows_o.append(jnp.sum(patch * w1, axis=(0, 1), keepdims=True
                                  ).reshape(1, CG))
    off1 = jnp.concatenate(rows_o, axis=0) + b1_ref[...]
    off1 = off1 * 0.5 * (1.0 + jax.lax.erf(off1 * (2.0 ** -0.5)))
    off = jnp.tanh(jax.lax.dot_general(w2_ref[...], off1, (((1,), (1,)), ((), ())),
                                       precision=_HI)) * OFFSET_SCALE

    ji = jax.lax.broadcasted_iota(jnp.int32, (1, KV), 1)
    gi = (ji // h2).astype(jnp.float32)
    gj = (ji % h2).astype(jnp.float32)
    vy = gi + off[0:1, :]
    vx = gj + off[1:2, :]
    gy = 2.0 * vy / max(h2 - 1, 1) - 1.0
    gx = 2.0 * vx / max(h2 - 1, 1) - 1.0

    ix = ((gx + 1.0) * Hs - 1.0) * 0.5
    iy = ((gy + 1.0) * Hs - 1.0) * 0.5
    ix0 = jnp.floor(ix)
    iy0 = jnp.floor(iy)
    wx1 = ix - ix0
    wx0 = 1.0 - wx1
    wy1 = iy - iy0
    wy0 = 1.0 - wy1
    piota = jax.lax.broadcasted_iota(jnp.int32, (N, KV), 0)
    S = jnp.zeros((N, KV), jnp.float32)
    for cx, cy, wgt in ((ix0, iy0, wx0 * wy0), (ix0 + 1.0, iy0, wx1 * wy0),
                        (ix0, iy0 + 1.0, wx0 * wy1), (ix0 + 1.0, iy0 + 1.0, wx1 * wy1)):
        valid = (cx >= 0) & (cx <= Hs - 1) & (cy >= 0) & (cy <= Hs - 1)
        idx = (jnp.clip(cy, 0, Hs - 1) * Hs + jnp.clip(cx, 0, Hs - 1)).astype(jnp.int32)
        wv_ = jnp.where(valid, wgt, 0.0)
        S = S + jnp.where(piota == idx, 1.0, 0.0) * wv_
    kv = jax.lax.dot(xr, S, precision=_HI)
    kb = jax.lax.dot(wk_ref[0], kv, precision=_HI)
    vb = jax.lax.dot(wv_ref[0], kv, precision=_HI)

    qx = jax.lax.broadcasted_iota(jnp.int32, (Hs, 1), 0).astype(jnp.float32)
    qxn = 2.0 * qx / max(Hs - 1, 1) - 1.0
    posx = qxn - gx
    fx = jnp.sign(posx) * jnp.log(jnp.abs(posx) + 1.0)
    R = QT * KV
    w1x = w1t_ref[0:1, :]
    w1y = w1t_ref[1:2, :]
    ax = fx[:, :, None] * w1x[None, :, :]

    bias_chunks = []
    for t in range(T):
        qy = (jax.lax.broadcasted_iota(jnp.int32, (rows, 1), 0)
              + t * rows).astype(jnp.float32)
        qyn = 2.0 * qy / max(Hs - 1, 1) - 1.0
        posy = qyn - gy
        fy = jnp.sign(posy) * jnp.log(jnp.abs(posy) + 1.0)
        ay = fy[:, :, None] * w1y[None, :, :] + b1c_ref[...][None, :, :]
        h1 = jax.nn.relu(ax[None, :, :, :] + ay[:, None, :, :])
        h1 = h1.reshape(R, CPB).astype(jnp.bfloat16)
        h2v = jax.lax.dot_general(h1, w2c_ref[...], (((1,), (1,)), ((), ())),
                                  preferred_element_type=jnp.float32) + b2c_ref[...]
        h2v = jax.nn.relu(h2v).astype(jnp.bfloat16)
        bc = jax.lax.dot_general(h2v, w3c_ref[...], (((1,), (0,)), ((), ())),
                                 preferred_element_type=jnp.float32)
        bias_chunks.append(bc[:, 0:1].reshape(QT, KV))
    bias = jnp.concatenate(bias_chunks, axis=0) + b3c_ref[0, 0]

    sim = jax.lax.dot(qT, kb, precision=_HI) + bias
    m = jnp.max(sim, axis=1, keepdims=True)
    e = jnp.exp(sim - m)
    attn = e / jnp.sum(e, axis=1, keepdims=True)
    out_ref[0, 0] = jax.lax.dot_general(vb, attn, (((1,), (1,)), ((), ())),
                                        precision=_HI)


def _stageB_kernel(oh_ref, wout_ref, bout_ref, y_ref):
    y_ref[0] = (jax.lax.dot(wout_ref[...], oh_ref[0], precision=_HI)
                + bout_ref[0][:, None])


def kernel(x, Wq, Wk, Wv, Woff1, boff1, Woff2, Wcpb1, bcpb1, Wcpb2, bcpb2,
           Wcpb3, bcpb3, Wout, bout):
    B, C, N = x.shape
    Hs = int(math.sqrt(N))
    pad = (OFF_K - DS) // 2
    h2 = (Hs + 2 * pad - OFF_K) // DS + 1
    KV = h2 * h2
    CPB = Wcpb1.shape[0]
    G = GROUPS
    scale = DIM_HEAD ** -0.5

    x4 = x.reshape(B, G, CG, N)
    wq3 = Wq[:, :, 0, 0].reshape(G, CG, CG)
    wk3 = Wk[:, :, 0, 0].reshape(G, CG, CG) * scale
    wv3 = Wv[:, :, 0, 0].reshape(G, CG, CG)
    w1 = Woff1[:, 0].transpose(1, 2, 0)
    b1 = boff1.reshape(1, CG)
    w2 = Woff2[:, :, 0, 0]
    w1t = Wcpb1.T
    b1c = bcpb1.reshape(1, CPB)
    b2c = bcpb2.reshape(1, CPB)
    w2b = Wcpb2.astype(jnp.bfloat16)
    w3 = jnp.zeros((CPB, 8), jnp.float32).at[:, 0].set(
        Wcpb3[0]).astype(jnp.bfloat16)
    b3 = bcpb3.reshape(1, 1)

    ROWS = 8
    outh = pl.pallas_call(
        functools.partial(_stageA_kernel, Hs=Hs, h2=h2, rows=ROWS),
        grid=(B, G),
        in_specs=[
            pl.BlockSpec((1, 1, CG, N), lambda b, g: (b, g, 0, 0)),
            pl.BlockSpec((1, CG, CG), lambda b, g: (g, 0, 0)),
            pl.BlockSpec((1, CG, CG), lambda b, g: (g, 0, 0)),
            pl.BlockSpec((1, CG, CG), lambda b, g: (g, 0, 0)),
            pl.BlockSpec((OFF_K, OFF_K, CG), lambda b, g: (0, 0, 0)),
            pl.BlockSpec((1, CG), lambda b, g: (0, 0)),
            pl.BlockSpec((2, CG), lambda b, g: (0, 0)),
            pl.BlockSpec((2, CPB), lambda b, g: (0, 0)),
            pl.BlockSpec((1, CPB), lambda b, g: (0, 0)),
            pl.BlockSpec((CPB, CPB), lambda b, g: (0, 0)),
            pl.BlockSpec((1, CPB), lambda b, g: (0, 0)),
            pl.BlockSpec((CPB, 8), lambda b, g: (0, 0)),
            pl.BlockSpec((1, 1), lambda b, g: (0, 0)),
        ],
        out_specs=pl.BlockSpec((1, 1, CG, N), lambda b, g: (b, g, 0, 0)),
        out_shape=jax.ShapeDtypeStruct((B, G, CG, N), jnp.float32),
        compiler_params=pltpu.CompilerParams(
            dimension_semantics=("parallel", "parallel")),
    )(x4, wq3, wk3, wv3, w1, b1, w2, w1t, b1c, w2b, b2c, w3, b3)

    oh = outh.reshape(B, DIM, N)
    wout2 = Wout[:, :, 0, 0]
    y = pl.pallas_call(
        _stageB_kernel,
        grid=(B,),
        in_specs=[
            pl.BlockSpec((1, DIM, N), lambda b: (b, 0, 0)),
            pl.BlockSpec((DIM, DIM), lambda b: (0, 0)),
            pl.BlockSpec((1, DIM), lambda b: (0, 0)),
        ],
        out_specs=pl.BlockSpec((1, DIM, N), lambda b: (b, 0, 0)),
        out_shape=jax.ShapeDtypeStruct((B, DIM, N), jnp.float32),
        compiler_params=pltpu.CompilerParams(
            dimension_semantics=("parallel",)),
    )(oh, wout2, bout.reshape(1, DIM))
    return y.reshape(B, DIM, Hs, Hs)


# speedup vs baseline: 4.9153x; 1.0099x over previous
import functools
import math

import jax
import jax.numpy as jnp
from jax.experimental import pallas as pl
from jax.experimental.pallas import tpu as pltpu

DIM = 768
GROUPS = 8
CG = DIM // GROUPS
HEADS = 8
DIM_HEAD = DIM // HEADS
OFFSET_SCALE = 4.0
OFF_K = 6
DS = 4

_HI = jax.lax.Precision.HIGHEST


def _stageA_kernel(x_ref, wq_ref, wk_ref, wv_ref, w1_ref, b1_ref, w2_ref,
                   w1t_ref, b1c_ref, w2c_ref, b2c_ref, w3c_ref, b3c_ref,
                   wo_ref, bo_ref, out_ref, *, Hs, h2, rows):
    N = Hs * Hs
    KV = h2 * h2
    CPB = w2c_ref.shape[0]
    QT = rows * Hs
    T = Hs // rows
    xr = x_ref[0, 0]

    qT = jax.lax.dot_general(xr, wq_ref[0], (((0,), (1,)), ((), ())),
                             precision=_HI)

    pad = (OFF_K - DS) // 2
    q3 = qT.reshape(Hs, Hs, CG)
    rpad_hi = DS * (h2 - 1) + OFF_K - pad - Hs
    zc = jnp.zeros((pad, Hs, CG), jnp.float32)
    zc2 = jnp.zeros((rpad_hi, Hs, CG), jnp.float32)
    qp = jnp.concatenate([zc, q3, zc2], axis=0)
    Hp = Hs + pad + rpad_hi
    zr = jnp.zeros((Hp, pad, CG), jnp.float32)
    zr2 = jnp.zeros((Hp, rpad_hi, CG), jnp.float32)
    qp = jnp.concatenate([zr, qp, zr2], axis=1)
    w1 = w1_ref[...]
    rows_o = []
    for oy in range(h2):
        for ox in range(h2):
            patch = qp[DS * oy:DS * oy + OFF_K, DS * ox:DS * ox + OFF_K, :]
            rows_o.append(jnp.sum(patch * w1, axis=(0, 1), keepdims=True
                                  ).reshape(1, CG))
    off1 = jnp.concatenate(rows_o, axis=0) + b1_ref[...]
    off1 = off1 * 0.5 * (1.0 + jax.lax.erf(off1 * (2.0 ** -0.5)))
    off = jnp.tanh(jax.lax.dot_general(w2_ref[...], off1, (((1,), (1,)), ((), ())),
                                       precision=_HI)) * OFFSET_SCALE

    ji = jax.lax.broadcasted_iota(jnp.int32, (1, KV), 1)
    gi = (ji // h2).astype(jnp.float32)
    gj = (ji % h2).astype(jnp.float32)
    vy = gi + off[0:1, :]
    vx = gj + off[1:2, :]
    gy = 2.0 * vy / max(h2 - 1, 1) - 1.0
    gx = 2.0 * vx / max(h2 - 1, 1) - 1.0

    ix = ((gx + 1.0) * Hs - 1.0) * 0.5
    iy = ((gy + 1.0) * Hs - 1.0) * 0.5
    ix0 = jnp.floor(ix)
    iy0 = jnp.floor(iy)
    wx1 = ix - ix0
    wx0 = 1.0 - wx1
    wy1 = iy - iy0
    wy0 = 1.0 - wy1
    piota = jax.lax.broadcasted_iota(jnp.int32, (N, KV), 0)
    S = jnp.zeros((N, KV), jnp.float32)
    for cx, cy, wgt in ((ix0, iy0, wx0 * wy0), (ix0 + 1.0, iy0, wx1 * wy0),
                        (ix0, iy0 + 1.0, wx0 * wy1), (ix0 + 1.0, iy0 + 1.0, wx1 * wy1)):
        valid = (cx >= 0) & (cx <= Hs - 1) & (cy >= 0) & (cy <= Hs - 1)
        idx = (jnp.clip(cy, 0, Hs - 1) * Hs + jnp.clip(cx, 0, Hs - 1)).astype(jnp.int32)
        wv_ = jnp.where(valid, wgt, 0.0)
        S = S + jnp.where(piota == idx, 1.0, 0.0) * wv_
    kv = jax.lax.dot(xr, S, precision=_HI)
    kb = jax.lax.dot(wk_ref[0], kv, precision=_HI)
    vb = jax.lax.dot(wv_ref[0], kv, precision=_HI)

    qx = jax.lax.broadcasted_iota(jnp.int32, (Hs, 1), 0).astype(jnp.float32)
    qxn = 2.0 * qx / max(Hs - 1, 1) - 1.0
    posx = qxn - gx
    fx = (jnp.sign(posx) * jnp.log(jnp.abs(posx) + 1.0)).astype(jnp.bfloat16)
    R = QT * KV
    w1x = w1t_ref[0:1, :]
    w1y = w1t_ref[1:2, :]
    ax = fx[:, :, None] * w1x[None, :, :]

    bias_chunks = []
    for t in range(T):
        qy = (jax.lax.broadcasted_iota(jnp.int32, (rows, 1), 0)
              + t * rows).astype(jnp.float32)
        qyn = 2.0 * qy / max(Hs - 1, 1) - 1.0
        posy = qyn - gy
        fy = (jnp.sign(posy) * jnp.log(jnp.abs(posy) + 1.0)).astype(jnp.bfloat16)
        ay = fy[:, :, None] * w1y[None, :, :] + b1c_ref[...][None, :, :]
        h1 = jax.nn.relu(ax[None, :, :, :] + ay[:, None, :, :])
        h1 = h1.reshape(R, CPB)
        h2v = jax.lax.dot_general(h1, w2c_ref[...], (((1,), (1,)), ((), ())),
                                  preferred_element_type=jnp.float32) + b2c_ref[...]
        h2v = jax.nn.relu(h2v).astype(jnp.bfloat16)
        bc = jax.lax.dot_general(h2v, w3c_ref[...], (((1,), (0,)), ((), ())),
                                 preferred_element_type=jnp.float32)
        bias_chunks.append(bc[:, 0:1].reshape(QT, KV))
    bias = jnp.concatenate(bias_chunks, axis=0) + b3c_ref[0, 0]

    sim = jax.lax.dot(qT, kb, precision=_HI) + bias
    m = jnp.max(sim, axis=1, keepdims=True)
    e = jnp.exp(sim - m)
    attn = e / jnp.sum(e, axis=1, keepdims=True)
    oh = jax.lax.dot_general(vb, attn, (((1,), (1,)), ((), ())),
                             precision=_HI)

    proj = jax.lax.dot(wo_ref[0], oh.astype(jnp.bfloat16),
                       preferred_element_type=jnp.float32)
    g = pl.program_id(1)

    @pl.when(g == 0)
    def _init():
        out_ref[0] = proj + bo_ref[...]

    @pl.when(g != 0)
    def _acc():
        out_ref[0] = out_ref[0] + proj


def kernel(x, Wq, Wk, Wv, Woff1, boff1, Woff2, Wcpb1, bcpb1, Wcpb2, bcpb2,
           Wcpb3, bcpb3, Wout, bout):
    B, C, N = x.shape
    Hs = int(math.sqrt(N))
    pad = (OFF_K - DS) // 2
    h2 = (Hs + 2 * pad - OFF_K) // DS + 1
    KV = h2 * h2
    CPB = Wcpb1.shape[0]
    G = GROUPS
    scale = DIM_HEAD ** -0.5

    x4 = x.reshape(B, G, CG, N)
    wq3 = Wq[:, :, 0, 0].reshape(G, CG, CG)
    wk3 = Wk[:, :, 0, 0].reshape(G, CG, CG) * scale
    wv3 = Wv[:, :, 0, 0].reshape(G, CG, CG)
    w1 = Woff1[:, 0].transpose(1, 2, 0)
    b1 = boff1.reshape(1, CG)
    w2 = Woff2[:, :, 0, 0]
    w1t = Wcpb1.T.astype(jnp.bfloat16)
    b1c = bcpb1.reshape(1, CPB).astype(jnp.bfloat16)
    b2c = bcpb2.reshape(1, CPB)
    w2b = Wcpb2.astype(jnp.bfloat16)
    w3 = jnp.zeros((CPB, 8), jnp.float32).at[:, 0].set(
        Wcpb3[0]).astype(jnp.bfloat16)
    b3 = bcpb3.reshape(1, 1)

    woutg = (Wout[:, :, 0, 0].reshape(DIM, G, CG).transpose(1, 0, 2)
             .astype(jnp.bfloat16))
    boutc = bout.reshape(DIM, 1)

    ROWS = 8
    y = pl.pallas_call(
        functools.partial(_stageA_kernel, Hs=Hs, h2=h2, rows=ROWS),
        grid=(B, G),
        in_specs=[
            pl.BlockSpec((1, 1, CG, N), lambda b, g: (b, g, 0, 0)),
            pl.BlockSpec((1, CG, CG), lambda b, g: (g, 0, 0)),
            pl.BlockSpec((1, CG, CG), lambda b, g: (g, 0, 0)),
            pl.BlockSpec((1, CG, CG), lambda b, g: (g, 0, 0)),
            pl.BlockSpec((OFF_K, OFF_K, CG), lambda b, g: (0, 0, 0)),
            pl.BlockSpec((1, CG), lambda b, g: (0, 0)),
            pl.BlockSpec((2, CG), lambda b, g: (0, 0)),
            pl.BlockSpec((2, CPB), lambda b, g: (0, 0)),
            pl.BlockSpec((1, CPB), lambda b, g: (0, 0)),
            pl.BlockSpec((CPB, CPB), lambda b, g: (0, 0)),
            pl.BlockSpec((1, CPB), lambda b, g: (0, 0)),
            pl.BlockSpec((CPB, 8), lambda b, g: (0, 0)),
            pl.BlockSpec((1, 1), lambda b, g: (0, 0)),
            pl.BlockSpec((1, DIM, CG), lambda b, g: (g, 0, 0)),
            pl.BlockSpec((DIM, 1), lambda b, g: (0, 0)),
        ],
        out_specs=pl.BlockSpec((1, DIM, N), lambda b, g: (b, 0, 0)),
        out_shape=jax.ShapeDtypeStruct((B, DIM, N), jnp.float32),
        compiler_params=pltpu.CompilerParams(
            dimension_semantics=("parallel", "arbitrary")),
    )(x4, wq3, wk3, wv3, w1, b1, w2, w1t, b1c, w2b, b2c, w3, b3, woutg, boutc)
    return y.reshape(B, DIM, Hs, Hs)


# speedup vs baseline: 5.3040x; 1.0791x over previous
import functools
import math

import jax
import jax.numpy as jnp
from jax.experimental import pallas as pl
from jax.experimental.pallas import tpu as pltpu

DIM = 768
GROUPS = 8
CG = DIM // GROUPS
HEADS = 8
DIM_HEAD = DIM // HEADS
OFFSET_SCALE = 4.0
OFF_K = 6
DS = 4

_HI = jax.lax.Precision.HIGHEST


def _stageA_kernel(x_ref, wq_ref, wk_ref, wv_ref, w1_ref, b1_ref, w2_ref,
                   w1t_ref, b1c_ref, w2c_ref, b2c_ref, w3c_ref, b3c_ref,
                   wo_ref, bo_ref, out_ref, *, Hs, h2, rows):
    N = Hs * Hs
    KV = h2 * h2
    CPB = w2c_ref.shape[0]
    QT = rows * Hs
    T = Hs // rows
    xr = x_ref[0, 0]

    qT = jax.lax.dot_general(xr, wq_ref[0], (((0,), (1,)), ((), ())))

    pad = (OFF_K - DS) // 2
    q3 = qT.reshape(Hs, Hs, CG)
    rpad_hi = DS * (h2 - 1) + OFF_K - pad - Hs
    zc = jnp.zeros((pad, Hs, CG), jnp.float32)
    zc2 = jnp.zeros((rpad_hi, Hs, CG), jnp.float32)
    qp = jnp.concatenate([zc, q3, zc2], axis=0)
    Hp = Hs + pad + rpad_hi
    zr = jnp.zeros((Hp, pad, CG), jnp.float32)
    zr2 = jnp.zeros((Hp, rpad_hi, CG), jnp.float32)
    qp = jnp.concatenate([zr, qp, zr2], axis=1)
    w1 = w1_ref[...]
    rows_o = []
    for oy in range(h2):
        for ox in range(h2):
            patch = qp[DS * oy:DS * oy + OFF_K, DS * ox:DS * ox + OFF_K, :]
            rows_o.append(jnp.sum(patch * w1, axis=(0, 1), keepdims=True
                                  ).reshape(1, CG))
    off1 = jnp.concatenate(rows_o, axis=0) + b1_ref[...]
    off1 = off1 * 0.5 * (1.0 + jax.lax.erf(off1 * (2.0 ** -0.5)))
    off = jnp.tanh(jax.lax.dot_general(w2_ref[...], off1, (((1,), (1,)), ((), ())),
                                       precision=_HI)) * OFFSET_SCALE

    ji = jax.lax.broadcasted_iota(jnp.int32, (1, KV), 1)
    gi = (ji // h2).astype(jnp.float32)
    gj = (ji % h2).astype(jnp.float32)
    vy = gi + off[0:1, :]
    vx = gj + off[1:2, :]
    gy = 2.0 * vy / max(h2 - 1, 1) - 1.0
    gx = 2.0 * vx / max(h2 - 1, 1) - 1.0

    ix = ((gx + 1.0) * Hs - 1.0) * 0.5
    iy = ((gy + 1.0) * Hs - 1.0) * 0.5
    ix0 = jnp.floor(ix)
    iy0 = jnp.floor(iy)
    wx1 = ix - ix0
    wx0 = 1.0 - wx1
    wy1 = iy - iy0
    wy0 = 1.0 - wy1
    piota = jax.lax.broadcasted_iota(jnp.int32, (N, KV), 0)
    S = jnp.zeros((N, KV), jnp.float32)
    for cx, cy, wgt in ((ix0, iy0, wx0 * wy0), (ix0 + 1.0, iy0, wx1 * wy0),
                        (ix0, iy0 + 1.0, wx0 * wy1), (ix0 + 1.0, iy0 + 1.0, wx1 * wy1)):
        valid = (cx >= 0) & (cx <= Hs - 1) & (cy >= 0) & (cy <= Hs - 1)
        idx = (jnp.clip(cy, 0, Hs - 1) * Hs + jnp.clip(cx, 0, Hs - 1)).astype(jnp.int32)
        wv_ = jnp.where(valid, wgt, 0.0)
        S = S + jnp.where(piota == idx, 1.0, 0.0) * wv_
    kv = jax.lax.dot(xr, S, precision=_HI)
    kb = jax.lax.dot(wk_ref[0], kv)
    vb = jax.lax.dot(wv_ref[0], kv)

    qx = jax.lax.broadcasted_iota(jnp.int32, (Hs, 1), 0).astype(jnp.float32)
    qxn = 2.0 * qx / max(Hs - 1, 1) - 1.0
    posx = qxn - gx
    fx = jnp.sign(posx) * jnp.log(jnp.abs(posx) + 1.0)
    R = QT * KV
    w1x = w1t_ref[0:1, :]
    w1y = w1t_ref[1:2, :]
    ax = fx[:, :, None] * w1x[None, :, :]

    bias_chunks = []
    for t in range(T):
        qy = (jax.lax.broadcasted_iota(jnp.int32, (rows, 1), 0)
              + t * rows).astype(jnp.float32)
        qyn = 2.0 * qy / max(Hs - 1, 1) - 1.0
        posy = qyn - gy
        fy = jnp.sign(posy) * jnp.log(jnp.abs(posy) + 1.0)
        ay = fy[:, :, None] * w1y[None, :, :] + b1c_ref[...][None, :, :]
        h1 = jax.nn.relu(ax[None, :, :, :] + ay[:, None, :, :])
        h1 = h1.reshape(R, CPB).astype(jnp.bfloat16)
        h2v = jax.lax.dot_general(h1, w2c_ref[...], (((1,), (1,)), ((), ())),
                                  preferred_element_type=jnp.float32) + b2c_ref[...]
        h2v = jax.nn.relu(h2v)
        bc = jax.lax.dot_general(h2v, w3c_ref[...], (((1,), (0,)), ((), ())),
                                 preferred_element_type=jnp.float32)
        bias_chunks.append(bc[:, 0:1].reshape(QT, KV))
    bias = jnp.concatenate(bias_chunks, axis=0) + b3c_ref[0, 0]

    sim = jax.lax.dot(qT, kb) + bias
    m = jnp.max(sim, axis=1, keepdims=True)
    e = jnp.exp(sim - m)
    attn = e / jnp.sum(e, axis=1, keepdims=True)
    oh = jax.lax.dot_general(vb, attn, (((1,), (1,)), ((), ())))

    proj = jax.lax.dot(wo_ref[0], oh.astype(jnp.bfloat16),
                       preferred_element_type=jnp.float32)
    g = pl.program_id(1)

    @pl.when(g == 0)
    def _init():
        out_ref[0] = proj + bo_ref[...]

    @pl.when(g != 0)
    def _acc():
        out_ref[0] = out_ref[0] + proj


def kernel(x, Wq, Wk, Wv, Woff1, boff1, Woff2, Wcpb1, bcpb1, Wcpb2, bcpb2,
           Wcpb3, bcpb3, Wout, bout):
    B, C, N = x.shape
    Hs = int(math.sqrt(N))
    pad = (OFF_K - DS) // 2
    h2 = (Hs + 2 * pad - OFF_K) // DS + 1
    KV = h2 * h2
    CPB = Wcpb1.shape[0]
    G = GROUPS
    scale = DIM_HEAD ** -0.5

    x4 = x.reshape(B, G, CG, N)
    wq3 = Wq[:, :, 0, 0].reshape(G, CG, CG)
    wk3 = Wk[:, :, 0, 0].reshape(G, CG, CG) * scale
    wv3 = Wv[:, :, 0, 0].reshape(G, CG, CG)
    w1 = Woff1[:, 0].transpose(1, 2, 0)
    b1 = boff1.reshape(1, CG)
    w2 = Woff2[:, :, 0, 0]
    w1t = Wcpb1.T
    b1c = bcpb1.reshape(1, CPB)
    b2c = bcpb2.reshape(1, CPB)
    w2b = Wcpb2.astype(jnp.bfloat16)
    w3 = jnp.zeros((CPB, 8), jnp.float32).at[:, 0].set(
        Wcpb3[0])
    b3 = bcpb3.reshape(1, 1)

    woutg = (Wout[:, :, 0, 0].reshape(DIM, G, CG).transpose(1, 0, 2)
             .astype(jnp.bfloat16))
    boutc = bout.reshape(DIM, 1)

    ROWS = 8
    y = pl.pallas_call(
        functools.partial(_stageA_kernel, Hs=Hs, h2=h2, rows=ROWS),
        grid=(B, G),
        in_specs=[
            pl.BlockSpec((1, 1, CG, N), lambda b, g: (b, g, 0, 0)),
            pl.BlockSpec((1, CG, CG), lambda b, g: (g, 0, 0)),
            pl.BlockSpec((1, CG, CG), lambda b, g: (g, 0, 0)),
            pl.BlockSpec((1, CG, CG), lambda b, g: (g, 0, 0)),
            pl.BlockSpec((OFF_K, OFF_K, CG), lambda b, g: (0, 0, 0)),
            pl.BlockSpec((1, CG), lambda b, g: (0, 0)),
            pl.BlockSpec((2, CG), lambda b, g: (0, 0)),
            pl.BlockSpec((2, CPB), lambda b, g: (0, 0)),
            pl.BlockSpec((1, CPB), lambda b, g: (0, 0)),
            pl.BlockSpec((CPB, CPB), lambda b, g: (0, 0)),
            pl.BlockSpec((1, CPB), lambda b, g: (0, 0)),
            pl.BlockSpec((CPB, 8), lambda b, g: (0, 0)),
            pl.BlockSpec((1, 1), lambda b, g: (0, 0)),
            pl.BlockSpec((1, DIM, CG), lambda b, g: (g, 0, 0)),
            pl.BlockSpec((DIM, 1), lambda b, g: (0, 0)),
        ],
        out_specs=pl.BlockSpec((1, DIM, N), lambda b, g: (b, 0, 0)),
        out_shape=jax.ShapeDtypeStruct((B, DIM, N), jnp.float32),
        compiler_params=pltpu.CompilerParams(
            dimension_semantics=("parallel", "arbitrary")),
    )(x4, wq3, wk3, wv3, w1, b1, w2, w1t, b1c, w2b, b2c, w3, b3, woutg, boutc)
    return y.reshape(B, DIM, Hs, Hs)
